# baseline (speedup 1.0000x reference)
import functools

import jax
import jax.numpy as jnp
from jax import lax
from jax.experimental import pallas as pl
from jax.experimental.pallas import tpu as pltpu
from jax.experimental.pallas import tpu_sc as plsc

N = 100000
E = 1600000
NWORK = 32
KCH = 8

NCH2 = 98
EP2_W = KCH * 128 * NCH2
EPAD2 = NWORK * EP2_W


def _make_gather(D, nch, rows_w):
    mesh = plsc.VectorSubcoreMesh(core_axis_name="c", subcore_axis_name="s")

    @functools.partial(
        pl.kernel,
        out_type=jax.ShapeDtypeStruct((rows_w * NWORK * 128, D), jnp.float32),
        mesh=mesh,
        compiler_params=pltpu.CompilerParams(use_tc_tiling_on_sc=False),
        scratch_types=[
            pltpu.VMEM((KCH, 128), jnp.int32),
            pltpu.VMEM((KCH * 128, D), jnp.float32),
            pltpu.SemaphoreType.DMA,
        ],
    )
    def gk(idx_hbm, table_hbm, out_hbm, idxb, rowsb, sem):
        c = lax.axis_index("c")
        s = lax.axis_index("s")
        wid = s * 2 + c

        def body(j, carry):
            rowbase = wid * rows_w + j * KCH
            pltpu.sync_copy(idx_hbm.at[pl.ds(rowbase, KCH)], idxb)
            descs = []
            for jj in range(KCH):
                descs.append(
                    pltpu.async_copy(
                        table_hbm.at[idxb.at[jj]],
                        rowsb.at[pl.ds(jj * 128, 128)],
                        sem,
                    )
                )
            for d in descs:
                d.wait()
            pltpu.sync_copy(rowsb, out_hbm.at[pl.ds(rowbase * 128, KCH * 128)])
            return carry

        lax.fori_loop(0, nch, body, 0)

    return gk


_gather2 = _make_gather(16, NCH2, KCH * NCH2)


def _leaky(v):
    return jnp.where(v >= 0, v, 0.2 * v)


def kernel(x, edge_index, edge_attr, W1, as1, ad1, ae1, We1, b1, Wm1, bm1, Wm2, bm2, Wm3, bm3, Wm4, bm4, W2, as2, ad2, ae2, We2, b2):
    src0 = edge_index[0]
    dst0 = edge_index[1]
    keep = src0 != dst0
    mf = keep.astype(jnp.float32)
    idx2 = jnp.concatenate([src0, dst0, jnp.zeros((EPAD2 - 2 * E,), jnp.int32)]).reshape(EPAD2 // 128, 128)

    eaT = edge_attr.T

    xl = x @ W1
    rows = _gather2(idx2, xl)
    rT = rows.T
    xl_srcT, xl_dstT = rT[:, :E], rT[:, E:2 * E]
    sa_n = xl @ as1
    sd_n = xl @ ad1
    wa1 = We1 @ ae1
    alpha_e = as1 @ xl_srcT + ad1 @ xl_dstT + wa1 @ eaT
    p_e = jnp.where(keep, jnp.exp(_leaky(alpha_e)), 0.0)
    sc_inT = jnp.concatenate(
        [mf[None], eaT * mf[None], p_e[None], p_e[None] * xl_srcT], axis=0)
    seg = jax.ops.segment_sum(sc_inT.T, dst0, num_segments=N)
    cnt = seg[:, 0]
    loop_attr = seg[:, 1:4] / jnp.maximum(cnt, 1.0)[:, None]
    p_l = jnp.exp(_leaky(sa_n + sd_n + loop_attr @ wa1))
    asum = seg[:, 4] + p_l
    x1 = (seg[:, 5:] + p_l[:, None] * xl) / (asum + 1e-16)[:, None] + b1

    rows1 = _gather2(idx2, x1)
    r1T = rows1.T
    x1sT, x1dT = r1T[:, :E], r1T[:, E:2 * E]
    hT = jax.nn.relu(Wm1[:16].T @ x1sT + Wm1[16:19].T @ eaT + Wm1[19:].T @ x1dT + bm1[:, None])
    hT = jax.nn.relu(Wm2.T @ hT + bm2[:, None])
    hT = jax.nn.relu(Wm3.T @ hT + bm3[:, None])
    ea2T = Wm4.T @ hT + bm4[:, None]

    xl2 = x1 @ W2
    sa2_n = xl2 @ as2
    sd2_n = xl2 @ ad2
    wa2 = We2 @ ae2
    alpha2 = (W2 @ as2) @ x1sT + (W2 @ ad2) @ x1dT + wa2 @ ea2T
    p2 = jnp.where(keep, jnp.exp(_leaky(alpha2)), 0.0)
    xl2_srcT = W2.T @ x1sT
    sc2_inT = jnp.concatenate(
        [ea2T * mf[None], p2[None], p2[None] * xl2_srcT], axis=0)
    seg2 = jax.ops.segment_sum(sc2_inT.T, dst0, num_segments=N)
    loop2 = seg2[:, :4] / jnp.maximum(cnt, 1.0)[:, None]
    p2_l = jnp.exp(_leaky(sa2_n + sd2_n + loop2 @ wa2))
    asum2 = seg2[:, 4] + p2_l
    x2 = (seg2[:, 5:] + p2_l[:, None] * xl2) / (asum2 + 1e-16)[:, None] + b2

    return jax.nn.log_softmax(x2, axis=1), jax.nn.log_softmax(ea2T, axis=0).T


# speedup vs baseline: 1.0494x; 1.0494x over previous
import functools

import jax
import jax.numpy as jnp
from jax import lax
from jax.experimental import pallas as pl
from jax.experimental.pallas import tpu as pltpu
from jax.experimental.pallas import tpu_sc as plsc

N = 100000
E = 1600000
NWORK = 32
KCH = 4

NCH2 = 196
EP2_W = KCH * 128 * NCH2
EPAD2 = NWORK * EP2_W


def _make_gather(D, nch, rows_w):
    mesh = plsc.VectorSubcoreMesh(core_axis_name="c", subcore_axis_name="s")

    @functools.partial(
        pl.kernel,
        out_type=jax.ShapeDtypeStruct((rows_w * NWORK * 128, D), jnp.float32),
        mesh=mesh,
        compiler_params=pltpu.CompilerParams(use_tc_tiling_on_sc=False),
        scratch_types=[
            pltpu.VMEM((KCH, 128), jnp.int32),
            pltpu.VMEM((KCH * 128, D), jnp.float32),
            pltpu.SemaphoreType.DMA,
        ],
    )
    def gk(idx_hbm, table_hbm, out_hbm, idxb, rowsb, sem):
        c = lax.axis_index("c")
        s = lax.axis_index("s")
        wid = s * 2 + c

        def body(j, carry):
            rowbase = wid * rows_w + j * KCH
            pltpu.sync_copy(idx_hbm.at[pl.ds(rowbase, KCH)], idxb)
            descs = []
            for jj in range(KCH):
                descs.append(
                    pltpu.async_copy(
                        table_hbm.at[idxb.at[jj]],
                        rowsb.at[pl.ds(jj * 128, 128)],
                        sem,
                    )
                )
            for d in descs:
                d.wait()
            pltpu.sync_copy(rowsb, out_hbm.at[pl.ds(rowbase * 128, KCH * 128)])
            return carry

        lax.fori_loop(0, nch, body, 0)

    return gk


_gather2 = _make_gather(16, NCH2, KCH * NCH2)


def _leaky(v):
    return jnp.where(v >= 0, v, 0.2 * v)


def kernel(x, edge_index, edge_attr, W1, as1, ad1, ae1, We1, b1, Wm1, bm1, Wm2, bm2, Wm3, bm3, Wm4, bm4, W2, as2, ad2, ae2, We2, b2):
    src0 = edge_index[0]
    dst0 = edge_index[1]
    keep = src0 != dst0
    mf = keep.astype(jnp.float32)
    idx2 = jnp.concatenate([src0, dst0, jnp.zeros((EPAD2 - 2 * E,), jnp.int32)]).reshape(EPAD2 // 128, 128)

    eaT = edge_attr.T

    rows = _gather2(idx2, x)
    rT = rows.T
    xsT, xdT = rT[:, :E], rT[:, E:2 * E]
    xl = x @ W1
    sa_n = xl @ as1
    sd_n = xl @ ad1
    wa1 = We1 @ ae1
    alpha_e = (W1 @ as1) @ xsT + (W1 @ ad1) @ xdT + wa1 @ eaT
    p_e = jnp.where(keep, jnp.exp(_leaky(alpha_e)), 0.0)
    xl_srcT = W1.T @ xsT
    sc_inT = jnp.concatenate(
        [mf[None], eaT * mf[None], p_e[None], p_e[None] * xl_srcT], axis=0)
    seg = jax.ops.segment_sum(sc_inT.T, dst0, num_segments=N)
    cnt = seg[:, 0]
    loop_attr = seg[:, 1:4] / jnp.maximum(cnt, 1.0)[:, None]
    p_l = jnp.exp(_leaky(sa_n + sd_n + loop_attr @ wa1))
    asum = seg[:, 4] + p_l
    x1 = (seg[:, 5:] + p_l[:, None] * xl) / (asum + 1e-16)[:, None] + b1

    rows1 = _gather2(idx2, x1)
    r1T = rows1.T
    x1sT, x1dT = r1T[:, :E], r1T[:, E:2 * E]
    hT = jax.nn.relu(Wm1[:16].T @ x1sT + Wm1[16:19].T @ eaT + Wm1[19:].T @ x1dT + bm1[:, None])
    hT = jax.nn.relu(Wm2.T @ hT + bm2[:, None])
    hT = jax.nn.relu(Wm3.T @ hT + bm3[:, None])
    ea2T = Wm4.T @ hT + bm4[:, None]

    xl2 = x1 @ W2
    sa2_n = xl2 @ as2
    sd2_n = xl2 @ ad2
    wa2 = We2 @ ae2
    alpha2 = (W2 @ as2) @ x1sT + (W2 @ ad2) @ x1dT + wa2 @ ea2T
    p2 = jnp.where(keep, jnp.exp(_leaky(alpha2)), 0.0)
    xl2_srcT = W2.T @ x1sT
    sc2_inT = jnp.concatenate(
        [ea2T * mf[None], p2[None], p2[None] * xl2_srcT], axis=0)
    seg2 = jax.ops.segment_sum(sc2_inT.T, dst0, num_segments=N)
    loop2 = seg2[:, :4] / jnp.maximum(cnt, 1.0)[:, None]
    p2_l = jnp.exp(_leaky(sa2_n + sd2_n + loop2 @ wa2))
    asum2 = seg2[:, 4] + p2_l
    x2 = (seg2[:, 5:] + p2_l[:, None] * xl2) / (asum2 + 1e-16)[:, None] + b2

    return jax.nn.log_softmax(x2, axis=1), jax.nn.log_softmax(ea2T, axis=0).T
